# baseline (speedup 1.0000x reference)
import functools

import jax
import jax.numpy as jnp
from jax import lax
from jax.experimental import pallas as pl
from jax.experimental.pallas import tpu as pltpu
from jax.experimental.pallas import tpu_sc as plsc

_N = 1000
_D = 1024
_R = 16
_TOPK = 50
_KPAD = 64
_BETA2 = 0.04
_TAU = 0.01
_SCALING = 2.0
_BROW = _D * _R // 128


def _routing_body(q_ref, c_ref, w_ref, ria_ref, rib_ref):
    q = q_ref[...]
    C = c_ref[...]
    qnorm = jnp.sqrt(jnp.sum(q * q)) + 1e-9
    ones = jnp.ones((1, _D), jnp.float32)
    dn = (((1,), (1,)), ((), ()))
    dots = lax.dot_general(q, C, dn, preferred_element_type=jnp.float32)
    cn2 = lax.dot_general(ones, C * C, dn, preferred_element_type=jnp.float32)
    cnorm = jnp.sqrt(cn2) + 1e-9
    sim = dots / (qnorm * cnorm * _BETA2)
    m0 = jnp.max(sim)
    e = jnp.exp(sim - m0)
    probs = e / jnp.sum(e)
    probs = jnp.where(probs >= _TAU, probs, 0.0)

    lane = lax.broadcasted_iota(jnp.int32, (1, _N), 1)
    col_t = lax.broadcasted_iota(jnp.int32, (64, 1), 0)

    def body(t, carry):
        P, wcol, icol, den = carry
        m = jnp.max(P)
        am = jnp.min(jnp.where(P == m, lane, jnp.int32(1 << 30)))
        den = den + m
        wcol = wcol + jnp.where(col_t == t, m, 0.0)
        icol = icol + jnp.where(col_t == t, am, 0)
        P = jnp.where(lane == am, -1.0, P)
        return P, wcol, icol, den

    init = (probs,
            jnp.zeros((64, 1), jnp.float32),
            jnp.zeros((64, 1), jnp.int32),
            jnp.float32(0.0))
    _, wcol, icol, den = lax.fori_loop(0, _TOPK, body, init)

    w_ref[...] = jnp.zeros((64, _R), jnp.float32) + wcol / (den + 1e-9) * _SCALING
    ria_ref[...] = icol * _R + lax.broadcasted_iota(jnp.int32, (64, _R), 1)
    rib_ref[...] = icol * _BROW + lax.broadcasted_iota(jnp.int32, (64, _BROW), 1)


_routing = pl.pallas_call(
    _routing_body,
    out_shape=[
        jax.ShapeDtypeStruct((64, _R), jnp.float32),
        jax.ShapeDtypeStruct((64, _R), jnp.int32),
        jax.ShapeDtypeStruct((64, _BROW), jnp.int32),
    ],
)


_A_ROWS_PER_W = (_KPAD * _R) // 32
_B_ROWS_PER_W = (_KPAD * _BROW) // 32
_B_CHUNKS = _B_ROWS_PER_W // 128


@functools.cache
def _make_sc_gather():
    return pl.kernel(
        _sc_gather_body,
        mesh=plsc.VectorSubcoreMesh(core_axis_name="c", subcore_axis_name="s"),
        out_type=[
            jax.ShapeDtypeStruct((_KPAD * _R, _D), jnp.float32),
            jax.ShapeDtypeStruct((_KPAD * _BROW, 128), jnp.float32),
        ],
        scratch_types=[
            pltpu.VMEM((_A_ROWS_PER_W,), jnp.int32),
            pltpu.VMEM((_A_ROWS_PER_W, _D), jnp.float32),
            pltpu.VMEM((_B_CHUNKS, 128), jnp.int32),
            pltpu.VMEM((_B_ROWS_PER_W, 128), jnp.float32),
            pltpu.SemaphoreType.DMA,
            pltpu.SemaphoreType.DMA,
        ],
    )


def _sc_gather_body(a2d, b2d, ria_hbm, rib_hbm, a_out, b_out,
                    ria_v, abuf, rib_v, bbuf, asem, bsem):
    wid = lax.axis_index("s") * 2 + lax.axis_index("c")
    pltpu.sync_copy(ria_hbm.at[pl.ds(wid * _A_ROWS_PER_W, _A_ROWS_PER_W)], ria_v)
    pltpu.sync_copy(rib_hbm.at[pl.ds(wid * _B_CHUNKS, _B_CHUNKS)], rib_v)

    a_copy = pltpu.make_async_copy(a2d.at[ria_v], abuf, asem)
    a_copy.start()
    b_copies = [
        pltpu.make_async_copy(b2d.at[rib_v.at[c]],
                              bbuf.at[pl.ds(c * 128, 128)], bsem)
        for c in range(_B_CHUNKS)
    ]
    for bc in b_copies:
        bc.start()
    a_copy.wait()
    pltpu.sync_copy(abuf, a_out.at[pl.ds(wid * _A_ROWS_PER_W, _A_ROWS_PER_W)])
    for bc in b_copies:
        bc.wait()
    pltpu.sync_copy(bbuf, b_out.at[pl.ds(wid * _B_ROWS_PER_W, _B_ROWS_PER_W)])


def _merge_body(b_ref, a_ref, w_ref, base_ref, o_ref):
    scaled = (a_ref[...] * w_ref[...]).reshape(_KPAD * _R, _D)
    bcat = jnp.concatenate([b_ref[k] for k in range(_KPAD)], axis=1)
    o_ref[...] = base_ref[...] + lax.dot(
        bcat, scaled, preferred_element_type=jnp.float32)


_merge = pl.pallas_call(
    _merge_body,
    out_shape=jax.ShapeDtypeStruct((_D, _D), jnp.float32),
)


def kernel(q, corpus, A_all, B_all, W_base):
    w64, ria, rib = _routing(q, corpus)
    a2d = A_all.reshape(_N * _R, _D)
    b2d = B_all.reshape(_N * _BROW, 128)
    acat, bg = _make_sc_gather()(a2d, b2d, ria.reshape(-1), rib)
    bg3 = bg.reshape(_KPAD, _D, _R)
    as3 = acat.reshape(_KPAD, _R, _D)
    w3 = w64.reshape(_KPAD, _R, 1)
    return _merge(bg3, as3, w3, W_base)


# speedup vs baseline: 8.1642x; 8.1642x over previous
import functools

import jax
import jax.numpy as jnp
from jax import lax
from jax.experimental import pallas as pl
from jax.experimental.pallas import tpu as pltpu
from jax.experimental.pallas import tpu_sc as plsc

_N = 1000
_D = 1024
_R = 16
_TOPK = 50
_KPAD = 64
_BETA2 = 0.04
_TAU = 0.01
_SCALING = 2.0


def _routing_body(q_ref, c_ref, w_ref, ria_ref):
    q = q_ref[...]
    C = c_ref[...]
    qnorm = jnp.sqrt(jnp.sum(q * q)) + 1e-9
    ones = jnp.ones((1, _D), jnp.float32)
    dn = (((1,), (1,)), ((), ()))
    dots = lax.dot_general(q, C, dn, preferred_element_type=jnp.float32)
    cn2 = lax.dot_general(ones, C * C, dn, preferred_element_type=jnp.float32)
    cnorm = jnp.sqrt(cn2) + 1e-9
    sim = dots / (qnorm * cnorm * _BETA2)
    m0 = jnp.max(sim)
    e = jnp.exp(sim - m0)
    probs = e / jnp.sum(e)
    probs = jnp.where(probs >= _TAU, probs, 0.0)

    lane = lax.broadcasted_iota(jnp.int32, (1, _N), 1)
    col_t = lax.broadcasted_iota(jnp.int32, (64, 1), 0)

    def body(t, carry):
        P, wcol, icol, den = carry
        m = jnp.max(P)
        am = jnp.min(jnp.where(P == m, lane, jnp.int32(1 << 30)))
        den = den + m
        wcol = wcol + jnp.where(col_t == t, m, 0.0)
        icol = icol + jnp.where(col_t == t, am, 0)
        P = jnp.where(lane == am, -1.0, P)
        return P, wcol, icol, den

    init = (probs,
            jnp.zeros((64, 1), jnp.float32),
            jnp.zeros((64, 1), jnp.int32),
            jnp.float32(0.0))
    _, wcol, icol, den = lax.fori_loop(0, _TOPK, body, init)

    w_ref[...] = jnp.zeros((64, _R), jnp.float32) + wcol / (den + 1e-9) * _SCALING
    ria_ref[...] = icol * _R + lax.broadcasted_iota(jnp.int32, (64, _R), 1)


_routing = pl.pallas_call(
    _routing_body,
    out_shape=[
        jax.ShapeDtypeStruct((64, _R), jnp.float32),
        jax.ShapeDtypeStruct((64, _R), jnp.int32),
    ],
)


_ROWS_PER_W = (_KPAD * _R) // 32


@functools.cache
def _make_sc_gather():
    return pl.kernel(
        _sc_gather_body,
        mesh=plsc.VectorSubcoreMesh(core_axis_name="c", subcore_axis_name="s"),
        out_type=[
            jax.ShapeDtypeStruct((_KPAD * _R, _D), jnp.float32),
            jax.ShapeDtypeStruct((_KPAD * _R, _D), jnp.float32),
        ],
        scratch_types=[
            pltpu.VMEM((_ROWS_PER_W,), jnp.int32),
            pltpu.VMEM((_ROWS_PER_W, _D), jnp.float32),
            pltpu.VMEM((_ROWS_PER_W, _D), jnp.float32),
            pltpu.SemaphoreType.DMA,
            pltpu.SemaphoreType.DMA,
        ],
    )


def _sc_gather_body(a2d, b2d, ria_hbm, a_out, b_out,
                    ria_v, abuf, bbuf, asem, bsem):
    wid = lax.axis_index("s") * 2 + lax.axis_index("c")
    sl = pl.ds(wid * _ROWS_PER_W, _ROWS_PER_W)
    pltpu.sync_copy(ria_hbm.at[sl], ria_v)
    a_copy = pltpu.make_async_copy(a2d.at[ria_v], abuf, asem)
    b_copy = pltpu.make_async_copy(b2d.at[ria_v], bbuf, bsem)
    a_copy.start()
    b_copy.start()
    a_copy.wait()
    pltpu.sync_copy(abuf, a_out.at[sl])
    b_copy.wait()
    pltpu.sync_copy(bbuf, b_out.at[sl])


def _merge_body(b_ref, a_ref, w_ref, base_ref, o_ref):
    scaled = a_ref[...] * w_ref[...]
    dn = (((0,), (0,)), ((), ()))
    o_ref[...] = base_ref[...] + lax.dot_general(
        b_ref[...], scaled, dn, preferred_element_type=jnp.float32)


_merge = pl.pallas_call(
    _merge_body,
    out_shape=jax.ShapeDtypeStruct((_D, _D), jnp.float32),
)


def kernel(q, corpus, A_all, B_all, W_base):
    w64, ria = _routing(q, corpus)
    a2d = A_all.reshape(_N * _R, _D)
    b2d = jnp.swapaxes(B_all, 1, 2).reshape(_N * _R, _D)
    ag, bg = _make_sc_gather()(a2d, b2d, ria.reshape(-1))
    wrow = w64.reshape(_KPAD * _R, 1)
    return _merge(bg, ag, wrow, W_base)


# speedup vs baseline: 8.2476x; 1.0102x over previous
import functools

import jax
import jax.numpy as jnp
from jax import lax
from jax.experimental import pallas as pl
from jax.experimental.pallas import tpu as pltpu
from jax.experimental.pallas import tpu_sc as plsc

_N = 1000
_D = 1024
_R = 16
_TOPK = 50
_KPAD = 64
_BETA2 = 0.04
_TAU = 0.01
_SCALING = 2.0


def _routing_body(q_ref, c_ref, w_ref, ria_ref):
    q = q_ref[...]
    C = c_ref[...]
    qnorm = jnp.sqrt(jnp.sum(q * q)) + 1e-9
    ones = jnp.ones((1, _D), jnp.float32)
    dn = (((1,), (1,)), ((), ()))
    dots = lax.dot_general(q, C, dn, preferred_element_type=jnp.float32)
    cn2 = lax.dot_general(ones, C * C, dn, preferred_element_type=jnp.float32)
    cnorm = jnp.sqrt(cn2) + 1e-9
    sim = dots / (qnorm * cnorm * _BETA2)
    m0 = jnp.max(sim)
    e = jnp.exp(sim - m0)
    probs = e / jnp.sum(e)
    probs = jnp.where(probs >= _TAU, probs, 0.0)

    probs_p = jnp.concatenate([probs, jnp.full((1, 24), -1.0, jnp.float32)], axis=1)
    P8 = jnp.concatenate([probs_p[:, i * 128:(i + 1) * 128] for i in range(8)], axis=0)
    fi = (lax.broadcasted_iota(jnp.int32, (8, 128), 0) * 128
          + lax.broadcasted_iota(jnp.int32, (8, 128), 1))
    l64 = lax.broadcasted_iota(jnp.int32, (1, 64), 1)
    col_t = lax.broadcasted_iota(jnp.int32, (64, 1), 0)

    def body(t, carry):
        P, wcol, irow, den = carry
        m = jnp.max(P, axis=(0, 1), keepdims=True)
        am = jnp.min(jnp.where(P == m, fi, jnp.int32(1 << 30)),
                     axis=(0, 1), keepdims=True)
        den = den + m
        wcol = wcol + jnp.where(col_t == t, m, 0.0)
        irow = irow + jnp.where(l64 == t, am, 0)
        P = jnp.where(fi == am, -1.0, P)
        return P, wcol, irow, den

    init = (P8,
            jnp.zeros((64, 1), jnp.float32),
            jnp.zeros((1, 64), jnp.int32),
            jnp.zeros((1, 1), jnp.float32))
    _, wcol, irow, den = lax.fori_loop(0, _TOPK, body, init)

    h_row = lax.broadcasted_iota(jnp.int32, (_KPAD * _R, 64), 0) // _R
    h_col = lax.broadcasted_iota(jnp.int32, (_KPAD * _R, 64), 1)
    H = jnp.where(h_row == h_col, 1.0, 0.0)
    wsc = wcol / (den + 1e-9) * _SCALING
    w_ref[...] = lax.dot_general(H, wsc, (((1,), (0,)), ((), ())),
                                 preferred_element_type=jnp.float32)

    i_row = lax.broadcasted_iota(jnp.int32, (8, 64), 0)
    i_col = lax.broadcasted_iota(jnp.int32, (8, 64), 1)
    IR = jnp.where(i_col // 8 == i_row, 1.0, 0.0) * irow.astype(jnp.float32)
    s_row = lax.broadcasted_iota(jnp.int32, (64, 128), 0)
    s_col = lax.broadcasted_iota(jnp.int32, (64, 128), 1)
    S = jnp.where(s_row % 8 == s_col // _R, 1.0, 0.0)
    irep8 = lax.dot_general(IR, S, (((1,), (0,)), ((), ())),
                            preferred_element_type=jnp.float32)
    lane_r = lax.broadcasted_iota(jnp.int32, (8, 128), 1) % _R
    ria_ref[...] = irep8.astype(jnp.int32) * _R + lane_r


_routing = pl.pallas_call(
    _routing_body,
    out_shape=[
        jax.ShapeDtypeStruct((_KPAD * _R, 1), jnp.float32),
        jax.ShapeDtypeStruct((8, 128), jnp.int32),
    ],
)


_ROWS_PER_W = (_KPAD * _R) // 32


@functools.cache
def _make_sc_gather():
    return pl.kernel(
        _sc_gather_body,
        mesh=plsc.VectorSubcoreMesh(core_axis_name="c", subcore_axis_name="s"),
        out_type=[
            jax.ShapeDtypeStruct((_KPAD * _R, _D), jnp.float32),
            jax.ShapeDtypeStruct((_KPAD * _R, _D), jnp.float32),
        ],
        scratch_types=[
            pltpu.VMEM((_ROWS_PER_W,), jnp.int32),
            pltpu.VMEM((_ROWS_PER_W, _D), jnp.float32),
            pltpu.VMEM((_ROWS_PER_W, _D), jnp.float32),
            pltpu.SemaphoreType.DMA,
            pltpu.SemaphoreType.DMA,
        ],
    )


def _sc_gather_body(a2d, b2d, ria_hbm, a_out, b_out,
                    ria_v, abuf, bbuf, asem, bsem):
    wid = lax.axis_index("s") * 2 + lax.axis_index("c")
    sl = pl.ds(wid * _ROWS_PER_W, _ROWS_PER_W)
    pltpu.sync_copy(ria_hbm.at[sl], ria_v)
    a_copy = pltpu.make_async_copy(a2d.at[ria_v], abuf, asem)
    b_copy = pltpu.make_async_copy(b2d.at[ria_v], bbuf, bsem)
    a_copy.start()
    b_copy.start()
    a_copy.wait()
    pltpu.sync_copy(abuf, a_out.at[sl])
    b_copy.wait()
    pltpu.sync_copy(bbuf, b_out.at[sl])


_MBLK = 256


def _merge_body(b_ref, a_ref, w_ref, base_ref, o_ref):
    scaled = b_ref[...] * w_ref[...]
    dn = (((0,), (0,)), ((), ()))
    o_ref[...] = base_ref[...] + lax.dot_general(
        scaled, a_ref[...], dn, preferred_element_type=jnp.float32)


_merge = pl.pallas_call(
    _merge_body,
    grid=(_D // _MBLK,),
    in_specs=[
        pl.BlockSpec((_KPAD * _R, _MBLK), lambda i: (0, i)),
        pl.BlockSpec((_KPAD * _R, _D), lambda i: (0, 0)),
        pl.BlockSpec((_KPAD * _R, 1), lambda i: (0, 0)),
        pl.BlockSpec((_MBLK, _D), lambda i: (i, 0)),
    ],
    out_specs=pl.BlockSpec((_MBLK, _D), lambda i: (i, 0)),
    out_shape=jax.ShapeDtypeStruct((_D, _D), jnp.float32),
)


def kernel(q, corpus, A_all, B_all, W_base):
    wrow, ria8 = _routing(q, corpus)
    a2d = A_all.reshape(_N * _R, _D)
    b2d = jnp.swapaxes(B_all, 1, 2).reshape(_N * _R, _D)
    ag, bg = _make_sc_gather()(a2d, b2d, ria8.reshape(-1))
    return _merge(bg, ag, wrow, W_base)


# speedup vs baseline: 13.5139x; 1.6385x over previous
import functools

import jax
import jax.numpy as jnp
from jax import lax
from jax.experimental import pallas as pl
from jax.experimental.pallas import tpu as pltpu
from jax.experimental.pallas import tpu_sc as plsc

_N = 1000
_D = 1024
_R = 16
_TOPK = 50
_KPAD = 64
_BETA2 = 0.04
_TAU = 0.01
_SCALING = 2.0
_KSMALL = 4


def _routing_body(q_ref, c_ref, w_ref, ria_ref, cnt_ref):
    q = q_ref[...]
    C = c_ref[...]
    qnorm = jnp.sqrt(jnp.sum(q * q)) + 1e-9
    ones = jnp.ones((1, _D), jnp.float32)
    dn = (((1,), (1,)), ((), ()))
    dots = lax.dot_general(q, C, dn, preferred_element_type=jnp.float32)
    cn2 = lax.dot_general(ones, C * C, dn, preferred_element_type=jnp.float32)
    cnorm = jnp.sqrt(cn2) + 1e-9
    sim = dots / (qnorm * cnorm * _BETA2)
    m0 = jnp.max(sim)
    e = jnp.exp(sim - m0)
    probs = e / jnp.sum(e)
    probs = jnp.where(probs >= _TAU, probs, 0.0)

    cnt = jnp.sum(jnp.where(probs >= _TAU, 1, 0))
    cnt = jnp.minimum(cnt, _TOPK)

    probs_p = jnp.concatenate([probs, jnp.full((1, 24), -1.0, jnp.float32)], axis=1)
    P8 = jnp.concatenate([probs_p[:, i * 128:(i + 1) * 128] for i in range(8)], axis=0)
    fi = (lax.broadcasted_iota(jnp.int32, (8, 128), 0) * 128
          + lax.broadcasted_iota(jnp.int32, (8, 128), 1))
    l64 = lax.broadcasted_iota(jnp.int32, (1, 64), 1)
    col_t = lax.broadcasted_iota(jnp.int32, (64, 1), 0)

    def body(t, carry):
        P, wcol, irow, den = carry
        m = jnp.max(P, axis=(0, 1), keepdims=True)
        am = jnp.min(jnp.where(P == m, fi, jnp.int32(1 << 30)),
                     axis=(0, 1), keepdims=True)
        den = den + m
        wcol = wcol + jnp.where(col_t == t, m, 0.0)
        irow = irow + jnp.where(l64 == t, am, 0)
        P = jnp.where(fi == am, -1.0, P)
        return P, wcol, irow, den

    init = (P8,
            jnp.zeros((64, 1), jnp.float32),
            jnp.zeros((1, 64), jnp.int32),
            jnp.zeros((1, 1), jnp.float32))
    _, wcol, irow, den = lax.fori_loop(0, cnt, body, init)

    h_row = lax.broadcasted_iota(jnp.int32, (_KPAD * _R, 64), 0) // _R
    h_col = lax.broadcasted_iota(jnp.int32, (_KPAD * _R, 64), 1)
    H = jnp.where(h_row == h_col, 1.0, 0.0)
    wsc = wcol / (den + 1e-9) * _SCALING
    w_ref[...] = lax.dot_general(H, wsc, (((1,), (0,)), ((), ())),
                                 preferred_element_type=jnp.float32)

    i_row = lax.broadcasted_iota(jnp.int32, (8, 64), 0)
    i_col = lax.broadcasted_iota(jnp.int32, (8, 64), 1)
    IR = jnp.where(i_col // 8 == i_row, 1.0, 0.0) * irow.astype(jnp.float32)
    s_row = lax.broadcasted_iota(jnp.int32, (64, 128), 0)
    s_col = lax.broadcasted_iota(jnp.int32, (64, 128), 1)
    S = jnp.where(s_row % 8 == s_col // _R, 1.0, 0.0)
    irep8 = lax.dot_general(IR, S, (((1,), (0,)), ((), ())),
                            preferred_element_type=jnp.float32)
    lane_r = lax.broadcasted_iota(jnp.int32, (8, 128), 1) % _R
    ria_ref[...] = irep8.astype(jnp.int32) * _R + lane_r
    cnt_ref[...] = jnp.zeros((1, 128), jnp.int32) + cnt


_routing = pl.pallas_call(
    _routing_body,
    out_shape=[
        jax.ShapeDtypeStruct((_KPAD * _R, 1), jnp.float32),
        jax.ShapeDtypeStruct((8, 128), jnp.int32),
        jax.ShapeDtypeStruct((1, 128), jnp.int32),
    ],
)


_ROWS_PER_W = (_KPAD * _R) // 32


@functools.cache
def _make_sc_gather():
    return pl.kernel(
        _sc_gather_body,
        mesh=plsc.VectorSubcoreMesh(core_axis_name="c", subcore_axis_name="s"),
        out_type=[
            jax.ShapeDtypeStruct((_KPAD * _R, _D), jnp.float32),
            jax.ShapeDtypeStruct((_KPAD * _R, _D), jnp.float32),
        ],
        scratch_types=[
            pltpu.VMEM((16,), jnp.int32),
            pltpu.VMEM((_ROWS_PER_W,), jnp.int32),
            pltpu.VMEM((_ROWS_PER_W, _D), jnp.float32),
            pltpu.VMEM((_ROWS_PER_W, _D), jnp.float32),
            pltpu.SemaphoreType.DMA,
            pltpu.SemaphoreType.DMA,
        ],
    )


def _sc_gather_body(a2d, b2d, ria_hbm, cnt_hbm, a_out, b_out,
                    cv, ria_v, abuf, bbuf, asem, bsem):
    wid = lax.axis_index("s") * 2 + lax.axis_index("c")
    pltpu.sync_copy(cnt_hbm.at[pl.ds(0, 16)], cv)
    cnt = cv[...][0]

    @pl.when(wid * 2 < cnt)
    def _():
        sl = pl.ds(wid * _ROWS_PER_W, _ROWS_PER_W)
        pltpu.sync_copy(ria_hbm.at[sl], ria_v)
        a_copy = pltpu.make_async_copy(a2d.at[ria_v], abuf, asem)
        b_copy = pltpu.make_async_copy(b2d.at[ria_v], bbuf, bsem)
        a_copy.start()
        b_copy.start()
        a_copy.wait()
        pltpu.sync_copy(abuf, a_out.at[sl])
        b_copy.wait()
        pltpu.sync_copy(bbuf, b_out.at[sl])


_MBLK = 256


def _merge_body(ksize, b_ref, a_ref, w_ref, cnt_ref, base_ref, o_ref):
    cnt = cnt_ref[0, 0]
    krow = lax.broadcasted_iota(jnp.int32, (ksize, 1), 0)
    live = krow < cnt * _R
    scaled = jnp.where(live, b_ref[...] * w_ref[...], 0.0)
    amask = jnp.where(live, a_ref[...], 0.0)
    dn = (((0,), (0,)), ((), ()))
    o_ref[...] = base_ref[...] + lax.dot_general(
        scaled, amask, dn, preferred_element_type=jnp.float32)


def _make_merge(ksize):
    return pl.pallas_call(
        functools.partial(_merge_body, ksize),
        grid=(_D // _MBLK,),
        in_specs=[
            pl.BlockSpec((ksize, _MBLK), lambda i: (0, i)),
            pl.BlockSpec((ksize, _D), lambda i: (0, 0)),
            pl.BlockSpec((ksize, 1), lambda i: (0, 0)),
            pl.BlockSpec((1, 128), lambda i: (0, 0)),
            pl.BlockSpec((_MBLK, _D), lambda i: (i, 0)),
        ],
        out_specs=pl.BlockSpec((_MBLK, _D), lambda i: (i, 0)),
        out_shape=jax.ShapeDtypeStruct((_D, _D), jnp.float32),
    )


def kernel(q, corpus, A_all, B_all, W_base):
    wrow, ria8, cnt = _routing(q, corpus)
    a2d = A_all.reshape(_N * _R, _D)
    b2d = jnp.swapaxes(B_all, 1, 2).reshape(_N * _R, _D)
    ag, bg = _make_sc_gather()(a2d, b2d, ria8.reshape(-1), cnt.reshape(-1))
    c0 = cnt[0, 0]
    return lax.cond(
        c0 <= _KSMALL,
        lambda: _make_merge(_KSMALL * _R)(bg, ag, wrow, cnt, W_base),
        lambda: _make_merge(_KPAD * _R)(bg, ag, wrow, cnt, W_base),
    )


# speedup vs baseline: 14.0629x; 1.0406x over previous
import functools

import jax
import jax.numpy as jnp
from jax import lax
from jax.experimental import pallas as pl
from jax.experimental.pallas import tpu as pltpu
from jax.experimental.pallas import tpu_sc as plsc

_N = 1000
_D = 1024
_R = 16
_TOPK = 50
_KPAD = 64
_BETA2 = 0.04
_TAU = 0.01
_SCALING = 2.0
_KSMALL = 4


def _routing_body(q_ref, c_ref, w_ref, ria_ref, cnt_ref):
    q = q_ref[...]
    C = c_ref[...]
    qnorm = jnp.sqrt(jnp.sum(q * q)) + 1e-9
    ones = jnp.ones((1, _D), jnp.float32)
    dn = (((1,), (1,)), ((), ()))
    dots = lax.dot_general(q, C, dn, preferred_element_type=jnp.float32)
    cn2 = lax.dot_general(ones, C * C, dn, preferred_element_type=jnp.float32)
    cnorm = jnp.sqrt(cn2) + 1e-9
    sim = dots / (qnorm * cnorm * _BETA2)
    m0 = jnp.max(sim)
    e = jnp.exp(sim - m0)
    probs = e / jnp.sum(e)
    probs = jnp.where(probs >= _TAU, probs, 0.0)

    cnt = jnp.sum(jnp.where(probs >= _TAU, 1, 0))
    cnt = jnp.minimum(cnt, _TOPK)

    probs_p = jnp.concatenate([probs, jnp.full((1, 24), -1.0, jnp.float32)], axis=1)
    P8 = jnp.concatenate([probs_p[:, i * 128:(i + 1) * 128] for i in range(8)], axis=0)
    fi = (lax.broadcasted_iota(jnp.int32, (8, 128), 0) * 128
          + lax.broadcasted_iota(jnp.int32, (8, 128), 1))
    l64 = lax.broadcasted_iota(jnp.int32, (1, 64), 1)
    col_t = lax.broadcasted_iota(jnp.int32, (64, 1), 0)

    def body(t, carry):
        P, wcol, irow, den = carry
        m = jnp.max(P, axis=(0, 1), keepdims=True)
        am = jnp.min(jnp.where(P == m, fi, jnp.int32(1 << 30)),
                     axis=(0, 1), keepdims=True)
        den = den + m
        wcol = wcol + jnp.where(col_t == t, m, 0.0)
        irow = irow + jnp.where(l64 == t, am, 0)
        P = jnp.where(fi == am, -1.0, P)
        return P, wcol, irow, den

    init = (P8,
            jnp.zeros((64, 1), jnp.float32),
            jnp.zeros((1, 64), jnp.int32),
            jnp.zeros((1, 1), jnp.float32))
    _, wcol, irow, den = lax.fori_loop(0, cnt, body, init)

    h_row = lax.broadcasted_iota(jnp.int32, (_KPAD * _R, 64), 0) // _R
    h_col = lax.broadcasted_iota(jnp.int32, (_KPAD * _R, 64), 1)
    H = jnp.where(h_row == h_col, 1.0, 0.0)
    wsc = wcol / (den + 1e-9) * _SCALING
    w_ref[...] = lax.dot_general(H, wsc, (((1,), (0,)), ((), ())),
                                 preferred_element_type=jnp.float32)

    i_row = lax.broadcasted_iota(jnp.int32, (8, 64), 0)
    i_col = lax.broadcasted_iota(jnp.int32, (8, 64), 1)
    IR = jnp.where(i_col // 8 == i_row, 1.0, 0.0) * irow.astype(jnp.float32)
    s_row = lax.broadcasted_iota(jnp.int32, (64, 128), 0)
    s_col = lax.broadcasted_iota(jnp.int32, (64, 128), 1)
    S = jnp.where(s_row % 8 == s_col // _R, 1.0, 0.0)
    irep8 = lax.dot_general(IR, S, (((1,), (0,)), ((), ())),
                            preferred_element_type=jnp.float32)
    lane_r = lax.broadcasted_iota(jnp.int32, (8, 128), 1) % _R
    ria_ref[...] = irep8.astype(jnp.int32) * _R + lane_r
    cnt_ref[...] = jnp.zeros((1, 128), jnp.int32) + cnt


_routing = pl.pallas_call(
    _routing_body,
    out_shape=[
        jax.ShapeDtypeStruct((_KPAD * _R, 1), jnp.float32),
        jax.ShapeDtypeStruct((8, 128), jnp.int32),
        jax.ShapeDtypeStruct((1, 128), jnp.int32),
    ],
)


_ROWS_PER_W = (_KPAD * _R) // 16
_RCHUNK = 32


@functools.cache
def _make_sc_gather():
    return pl.kernel(
        _sc_gather_body,
        mesh=plsc.VectorSubcoreMesh(core_axis_name="c", subcore_axis_name="s",
                                    num_cores=1),
        out_type=[
            jax.ShapeDtypeStruct((_KPAD * _R, _D), jnp.float32),
            jax.ShapeDtypeStruct((_KPAD * _R, _D), jnp.float32),
        ],
        scratch_types=[
            pltpu.VMEM((16,), jnp.int32),
            pltpu.VMEM((_RCHUNK,), jnp.int32),
            pltpu.VMEM((_RCHUNK, _D), jnp.float32),
            pltpu.VMEM((_RCHUNK, _D), jnp.float32),
            pltpu.SemaphoreType.DMA,
            pltpu.SemaphoreType.DMA,
        ],
    )


def _sc_gather_body(a2d, b2d, ria_hbm, cnt_hbm, a_out, b_out,
                    cv, ria_v, abuf, bbuf, asem, bsem):
    wid = lax.axis_index("s")
    pltpu.sync_copy(cnt_hbm.at[pl.ds(0, 16)], cv)
    cnt = cv[...][0]

    for u in range(_ROWS_PER_W // _RCHUNK):
        @pl.when((wid * 4 + u * 2) * _R < cnt * _R)
        def _():
            sl = pl.ds(wid * _ROWS_PER_W + u * _RCHUNK, _RCHUNK)
            pltpu.sync_copy(ria_hbm.at[sl], ria_v)
            a_copy = pltpu.make_async_copy(a2d.at[ria_v], abuf, asem)
            b_copy = pltpu.make_async_copy(b2d.at[ria_v], bbuf, bsem)
            a_copy.start()
            b_copy.start()
            a_copy.wait()
            pltpu.sync_copy(abuf, a_out.at[sl])
            b_copy.wait()
            pltpu.sync_copy(bbuf, b_out.at[sl])


_MBLK = 256


def _merge_body(ksize, b_ref, a_ref, w_ref, cnt_ref, base_ref, o_ref):
    cnt = cnt_ref[0, 0]
    krow = lax.broadcasted_iota(jnp.int32, (ksize, 1), 0)
    live = krow < cnt * _R
    scaled = jnp.where(live, b_ref[...] * w_ref[...], 0.0)
    amask = jnp.where(live, a_ref[...], 0.0)
    dn = (((0,), (0,)), ((), ()))
    o_ref[...] = base_ref[...] + lax.dot_general(
        scaled, amask, dn, preferred_element_type=jnp.float32)


def _make_merge(ksize):
    return pl.pallas_call(
        functools.partial(_merge_body, ksize),
        grid=(_D // _MBLK,),
        in_specs=[
            pl.BlockSpec((ksize, _MBLK), lambda i: (0, i)),
            pl.BlockSpec((ksize, _D), lambda i: (0, 0)),
            pl.BlockSpec((ksize, 1), lambda i: (0, 0)),
            pl.BlockSpec((1, 128), lambda i: (0, 0)),
            pl.BlockSpec((_MBLK, _D), lambda i: (i, 0)),
        ],
        out_specs=pl.BlockSpec((_MBLK, _D), lambda i: (i, 0)),
        out_shape=jax.ShapeDtypeStruct((_D, _D), jnp.float32),
    )


def kernel(q, corpus, A_all, B_all, W_base):
    wrow, ria8, cnt = _routing(q, corpus)
    a2d = A_all.reshape(_N * _R, _D)
    b2d = jnp.swapaxes(B_all, 1, 2).reshape(_N * _R, _D)
    ag, bg = _make_sc_gather()(a2d, b2d, ria8.reshape(-1), cnt.reshape(-1))
    c0 = cnt[0, 0]
    return lax.cond(
        c0 <= _KSMALL,
        lambda: _make_merge(_KSMALL * _R)(bg, ag, wrow, cnt, W_base),
        lambda: _make_merge(_KPAD * _R)(bg, ag, wrow, cnt, W_base),
    )


# speedup vs baseline: 14.7179x; 1.0466x over previous
import functools

import jax
import jax.numpy as jnp
from jax import lax
from jax.experimental import pallas as pl
from jax.experimental.pallas import tpu as pltpu
from jax.experimental.pallas import tpu_sc as plsc

_N = 1000
_D = 1024
_R = 16
_TOPK = 50
_KPAD = 64
_BETA2 = 0.04
_TAU = 0.01
_SCALING = 2.0
_KSMALL = 4


def _routing_body(q_ref, c_ref, w_ref, ria_ref, cnt_ref):
    q = q_ref[...]
    C = c_ref[...]
    qnorm = jnp.sqrt(jnp.sum(q * q)) + 1e-9
    ones = jnp.ones((1, _D), jnp.float32)
    dn = (((1,), (1,)), ((), ()))
    dots = lax.dot_general(q, C, dn, preferred_element_type=jnp.float32)
    cn2 = lax.dot_general(ones, C * C, dn, preferred_element_type=jnp.float32)
    cnorm = jnp.sqrt(cn2) + 1e-9
    sim = dots / (qnorm * cnorm * _BETA2)
    m0 = jnp.max(sim)
    e = jnp.exp(sim - m0)
    probs = e / jnp.sum(e)
    probs = jnp.where(probs >= _TAU, probs, 0.0)

    cnt = jnp.sum(jnp.where(probs >= _TAU, 1, 0))
    cnt = jnp.minimum(cnt, _TOPK)

    probs_p = jnp.concatenate([probs, jnp.full((1, 24), -1.0, jnp.float32)], axis=1)
    P8 = jnp.concatenate([probs_p[:, i * 128:(i + 1) * 128] for i in range(8)], axis=0)
    fi = (lax.broadcasted_iota(jnp.int32, (8, 128), 0) * 128
          + lax.broadcasted_iota(jnp.int32, (8, 128), 1))
    l64 = lax.broadcasted_iota(jnp.int32, (1, 64), 1)
    col_t = lax.broadcasted_iota(jnp.int32, (64, 1), 0)

    def body(t, carry):
        P, wcol, irow, den = carry
        m = jnp.max(P, axis=(0, 1), keepdims=True)
        am = jnp.min(jnp.where(P == m, fi, jnp.int32(1 << 30)),
                     axis=(0, 1), keepdims=True)
        den = den + m
        wcol = wcol + jnp.where(col_t == t, m, 0.0)
        irow = irow + jnp.where(l64 == t, am, 0)
        P = jnp.where(fi == am, -1.0, P)
        return P, wcol, irow, den

    init = (P8,
            jnp.zeros((64, 1), jnp.float32),
            jnp.zeros((1, 64), jnp.int32),
            jnp.zeros((1, 1), jnp.float32))
    _, wcol, irow, den = lax.fori_loop(0, cnt, body, init)

    h_row = lax.broadcasted_iota(jnp.int32, (_KPAD * _R, 64), 0) // _R
    h_col = lax.broadcasted_iota(jnp.int32, (_KPAD * _R, 64), 1)
    H = jnp.where(h_row == h_col, 1.0, 0.0)
    wsc = wcol / (den + 1e-9) * _SCALING
    w_ref[...] = lax.dot_general(H, wsc, (((1,), (0,)), ((), ())),
                                 preferred_element_type=jnp.float32)

    i_row = lax.broadcasted_iota(jnp.int32, (8, 64), 0)
    i_col = lax.broadcasted_iota(jnp.int32, (8, 64), 1)
    IR = jnp.where(i_col // 8 == i_row, 1.0, 0.0) * irow.astype(jnp.float32)
    s_row = lax.broadcasted_iota(jnp.int32, (64, 128), 0)
    s_col = lax.broadcasted_iota(jnp.int32, (64, 128), 1)
    S = jnp.where(s_row % 8 == s_col // _R, 1.0, 0.0)
    irep8 = lax.dot_general(IR, S, (((1,), (0,)), ((), ())),
                            preferred_element_type=jnp.float32)
    lane_r = lax.broadcasted_iota(jnp.int32, (8, 128), 1) % _R
    ria_ref[...] = irep8.astype(jnp.int32) * _R + lane_r
    cnt_ref[...] = jnp.zeros((1, 128), jnp.int32) + cnt


_routing = pl.pallas_call(
    _routing_body,
    out_shape=[
        jax.ShapeDtypeStruct((_KPAD * _R, 1), jnp.float32),
        jax.ShapeDtypeStruct((8, 128), jnp.int32),
        jax.ShapeDtypeStruct((1, 128), jnp.int32),
    ],
)


_ROWS_PER_W = (_KPAD * _R) // 16
_RCHUNK = 32


@functools.cache
def _make_sc_gather():
    return pl.kernel(
        _sc_gather_body,
        mesh=plsc.VectorSubcoreMesh(core_axis_name="c", subcore_axis_name="s",
                                    num_cores=1),
        out_type=[
            jax.ShapeDtypeStruct((_KPAD * _R, _D), jnp.float32),
            jax.ShapeDtypeStruct((_KPAD * _R, _D), jnp.float32),
        ],
        scratch_types=[
            pltpu.VMEM((16,), jnp.int32),
            pltpu.VMEM((_RCHUNK,), jnp.int32),
            pltpu.VMEM((_RCHUNK, _D), jnp.float32),
            pltpu.VMEM((_RCHUNK, _D), jnp.float32),
            pltpu.SemaphoreType.DMA,
            pltpu.SemaphoreType.DMA,
        ],
    )


def _sc_gather_body(a2d, b2d, ria_hbm, cnt_hbm, a_out, b_out,
                    cv, ria_v, abuf, bbuf, asem, bsem):
    wid = lax.axis_index("s")
    pltpu.sync_copy(cnt_hbm.at[pl.ds(0, 16)], cv)
    cnt = cv[...][0]

    for u in range(_ROWS_PER_W // _RCHUNK):
        @pl.when((wid * 4 + u * 2) * _R < cnt * _R)
        def _():
            sl = pl.ds(wid * _ROWS_PER_W + u * _RCHUNK, _RCHUNK)
            pltpu.sync_copy(ria_hbm.at[sl], ria_v)
            a_copy = pltpu.make_async_copy(a2d.at[ria_v], abuf, asem)
            b_copy = pltpu.make_async_copy(b2d.at[ria_v], bbuf, bsem)
            a_copy.start()
            b_copy.start()
            a_copy.wait()
            pltpu.sync_copy(abuf, a_out.at[sl])
            b_copy.wait()
            pltpu.sync_copy(bbuf, b_out.at[sl])


_MBLK = 512


def _merge_body(ksize, b_ref, a_ref, w_ref, cnt_ref, base_ref, o_ref):
    cnt = cnt_ref[0, 0]
    krow = lax.broadcasted_iota(jnp.int32, (ksize, 1), 0)
    live = krow < cnt * _R
    scaled = jnp.where(live, b_ref[...] * w_ref[...], 0.0)
    amask = jnp.where(live, a_ref[...], 0.0)
    dn = (((0,), (0,)), ((), ()))
    o_ref[...] = base_ref[...] + lax.dot_general(
        scaled, amask, dn, preferred_element_type=jnp.float32)


def _make_merge(ksize):
    return pl.pallas_call(
        functools.partial(_merge_body, ksize),
        grid=(_D // _MBLK,),
        in_specs=[
            pl.BlockSpec((ksize, _MBLK), lambda i: (0, i)),
            pl.BlockSpec((ksize, _D), lambda i: (0, 0)),
            pl.BlockSpec((ksize, 1), lambda i: (0, 0)),
            pl.BlockSpec((1, 128), lambda i: (0, 0)),
            pl.BlockSpec((_MBLK, _D), lambda i: (i, 0)),
        ],
        out_specs=pl.BlockSpec((_MBLK, _D), lambda i: (i, 0)),
        out_shape=jax.ShapeDtypeStruct((_D, _D), jnp.float32),
    )


def kernel(q, corpus, A_all, B_all, W_base):
    wrow, ria8, cnt = _routing(q, corpus)
    a2d = A_all.reshape(_N * _R, _D)
    b2d = jnp.swapaxes(B_all, 1, 2).reshape(_N * _R, _D)
    ag, bg = _make_sc_gather()(a2d, b2d, ria8.reshape(-1), cnt.reshape(-1))
    c0 = cnt[0, 0]
    return lax.cond(
        c0 <= _KSMALL,
        lambda: _make_merge(_KSMALL * _R)(bg, ag, wrow, cnt, W_base),
        lambda: _make_merge(_KPAD * _R)(bg, ag, wrow, cnt, W_base),
    )


# speedup vs baseline: 18.8139x; 1.2783x over previous
import functools

import jax
import jax.numpy as jnp
from jax import lax
from jax.experimental import pallas as pl
from jax.experimental.pallas import tpu as pltpu
from jax.experimental.pallas import tpu_sc as plsc

_N = 1000
_D = 1024
_R = 16
_TOPK = 50
_KPAD = 64
_BETA2 = 0.04
_TAU = 0.01
_SCALING = 2.0
_KSMALL = 4


def _routing_body(q_ref, c_ref, w_ref, ria_ref, cnt_ref, idx_ref):
    q = q_ref[...]
    C = c_ref[...]
    qnorm = jnp.sqrt(jnp.sum(q * q)) + 1e-9
    ones = jnp.ones((1, _D), jnp.float32)
    dn = (((1,), (1,)), ((), ()))
    dots = lax.dot_general(q, C, dn, preferred_element_type=jnp.float32)
    cn2 = lax.dot_general(ones, C * C, dn, preferred_element_type=jnp.float32)
    cnorm = jnp.sqrt(cn2) + 1e-9
    sim = dots / (qnorm * cnorm * _BETA2)
    m0 = jnp.max(sim)
    e = jnp.exp(sim - m0)
    probs = e / jnp.sum(e)
    probs = jnp.where(probs >= _TAU, probs, 0.0)

    cnt = jnp.sum(jnp.where(probs >= _TAU, 1, 0))
    cnt = jnp.minimum(cnt, _TOPK)

    probs_p = jnp.concatenate([probs, jnp.full((1, 24), -1.0, jnp.float32)], axis=1)
    P8 = jnp.concatenate([probs_p[:, i * 128:(i + 1) * 128] for i in range(8)], axis=0)
    fi = (lax.broadcasted_iota(jnp.int32, (8, 128), 0) * 128
          + lax.broadcasted_iota(jnp.int32, (8, 128), 1))
    l64 = lax.broadcasted_iota(jnp.int32, (1, 64), 1)
    col_t = lax.broadcasted_iota(jnp.int32, (64, 1), 0)

    def body(t, carry):
        P, wcol, irow, den = carry
        m = jnp.max(P, axis=(0, 1), keepdims=True)
        am = jnp.min(jnp.where(P == m, fi, jnp.int32(1 << 30)),
                     axis=(0, 1), keepdims=True)
        den = den + m
        wcol = wcol + jnp.where(col_t == t, m, 0.0)
        irow = irow + jnp.where(l64 == t, am, 0)
        P = jnp.where(fi == am, -1.0, P)
        return P, wcol, irow, den

    init = (P8,
            jnp.zeros((64, 1), jnp.float32),
            jnp.zeros((1, 64), jnp.int32),
            jnp.zeros((1, 1), jnp.float32))
    _, wcol, irow, den = lax.fori_loop(0, cnt, body, init)

    h_row = lax.broadcasted_iota(jnp.int32, (_KPAD * _R, 64), 0) // _R
    h_col = lax.broadcasted_iota(jnp.int32, (_KPAD * _R, 64), 1)
    H = jnp.where(h_row == h_col, 1.0, 0.0)
    wsc = wcol / (den + 1e-9) * _SCALING
    w_ref[...] = lax.dot_general(H, wsc, (((1,), (0,)), ((), ())),
                                 preferred_element_type=jnp.float32)

    i_row = lax.broadcasted_iota(jnp.int32, (8, 64), 0)
    i_col = lax.broadcasted_iota(jnp.int32, (8, 64), 1)
    IR = jnp.where(i_col // 8 == i_row, 1.0, 0.0) * irow.astype(jnp.float32)
    s_row = lax.broadcasted_iota(jnp.int32, (64, 128), 0)
    s_col = lax.broadcasted_iota(jnp.int32, (64, 128), 1)
    S = jnp.where(s_row % 8 == s_col // _R, 1.0, 0.0)
    irep8 = lax.dot_general(IR, S, (((1,), (0,)), ((), ())),
                            preferred_element_type=jnp.float32)
    lane_r = lax.broadcasted_iota(jnp.int32, (8, 128), 1) % _R
    ria_ref[...] = irep8.astype(jnp.int32) * _R + lane_r
    cnt_ref[...] = jnp.zeros((1, 128), jnp.int32) + cnt
    idx_ref[...] = jnp.concatenate([irow, jnp.zeros((1, 64), jnp.int32)], axis=1)


_routing = pl.pallas_call(
    _routing_body,
    out_shape=[
        jax.ShapeDtypeStruct((_KPAD * _R, 1), jnp.float32),
        jax.ShapeDtypeStruct((8, 128), jnp.int32),
        jax.ShapeDtypeStruct((1, 128), jnp.int32),
        jax.ShapeDtypeStruct((1, 128), jnp.int32),
    ],
)


_ROWS_PER_W = (_KPAD * _R) // 16
_RCHUNK = 32


@functools.cache
def _make_sc_gather():
    return pl.kernel(
        _sc_gather_body,
        mesh=plsc.VectorSubcoreMesh(core_axis_name="c", subcore_axis_name="s",
                                    num_cores=1),
        out_type=[
            jax.ShapeDtypeStruct((_KPAD * _R, _D), jnp.float32),
            jax.ShapeDtypeStruct((_KPAD * _R, _D), jnp.float32),
        ],
        scratch_types=[
            pltpu.VMEM((16,), jnp.int32),
            pltpu.VMEM((_RCHUNK,), jnp.int32),
            pltpu.VMEM((_RCHUNK, _D), jnp.float32),
            pltpu.VMEM((_RCHUNK, _D), jnp.float32),
            pltpu.SemaphoreType.DMA,
            pltpu.SemaphoreType.DMA,
        ],
    )


def _sc_gather_body(a2d, b2d, ria_hbm, cnt_hbm, a_out, b_out,
                    cv, ria_v, abuf, bbuf, asem, bsem):
    wid = lax.axis_index("s")
    pltpu.sync_copy(cnt_hbm.at[pl.ds(0, 16)], cv)
    cnt = cv[...][0]

    for u in range(_ROWS_PER_W // _RCHUNK):
        @pl.when((wid * 4 + u * 2) * _R < cnt * _R)
        def _():
            sl = pl.ds(wid * _ROWS_PER_W + u * _RCHUNK, _RCHUNK)
            pltpu.sync_copy(ria_hbm.at[sl], ria_v)
            a_copy = pltpu.make_async_copy(a2d.at[ria_v], abuf, asem)
            b_copy = pltpu.make_async_copy(b2d.at[ria_v], bbuf, bsem)
            a_copy.start()
            b_copy.start()
            a_copy.wait()
            pltpu.sync_copy(abuf, a_out.at[sl])
            b_copy.wait()
            pltpu.sync_copy(bbuf, b_out.at[sl])


_MBLK = 512


def _merge_body(ksize, b_ref, a_ref, w_ref, cnt_ref, base_ref, o_ref):
    cnt = cnt_ref[0, 0]
    krow = lax.broadcasted_iota(jnp.int32, (ksize, 1), 0)
    live = krow < cnt * _R
    scaled = jnp.where(live, b_ref[...] * w_ref[...], 0.0)
    amask = jnp.where(live, a_ref[...], 0.0)
    dn = (((0,), (0,)), ((), ()))
    o_ref[...] = base_ref[...] + lax.dot_general(
        scaled, amask, dn, preferred_element_type=jnp.float32)


def _make_merge(ksize):
    return pl.pallas_call(
        functools.partial(_merge_body, ksize),
        grid=(_D // _MBLK,),
        in_specs=[
            pl.BlockSpec((ksize, _MBLK), lambda i: (0, i)),
            pl.BlockSpec((ksize, _D), lambda i: (0, 0)),
            pl.BlockSpec((ksize, 1), lambda i: (0, 0)),
            pl.BlockSpec((1, 128), lambda i: (0, 0)),
            pl.BlockSpec((_MBLK, _D), lambda i: (i, 0)),
        ],
        out_specs=pl.BlockSpec((_MBLK, _D), lambda i: (i, 0)),
        out_shape=jax.ShapeDtypeStruct((_D, _D), jnp.float32),
    )


def _small_body(idx_ref, cnt_ref, a2d_ref, b2d_ref, w_ref, base_ref, o_ref,
                ag_s, bg_s, sem):
    @pl.when(pl.program_id(0) == 0)
    def _():
        copies = []
        for s in range(_KSMALL):
            cid = idx_ref[0, s]
            copies.append(pltpu.make_async_copy(
                a2d_ref.at[pl.ds(cid * _R, _R)], ag_s.at[pl.ds(s * _R, _R)], sem))
            copies.append(pltpu.make_async_copy(
                b2d_ref.at[pl.ds(cid * _R, _R)], bg_s.at[pl.ds(s * _R, _R)], sem))
        for cp in copies:
            cp.start()
        for cp in copies:
            cp.wait()

    i = pl.program_id(0)
    cnt = cnt_ref[0, 0]
    krow = lax.broadcasted_iota(jnp.int32, (_KSMALL * _R, 1), 0)
    live = krow < cnt * _R
    bg_blk = bg_s[:, pl.ds(i * _MBLK, _MBLK)]
    scaled = jnp.where(live, bg_blk * w_ref[...], 0.0)
    amask = jnp.where(live, ag_s[...], 0.0)
    dn = (((0,), (0,)), ((), ()))
    o_ref[...] = base_ref[...] + lax.dot_general(
        scaled, amask, dn, preferred_element_type=jnp.float32)


_small_merge = pl.pallas_call(
    _small_body,
    grid=(_D // _MBLK,),
    in_specs=[
        pl.BlockSpec(memory_space=pltpu.SMEM),
        pl.BlockSpec(memory_space=pltpu.SMEM),
        pl.BlockSpec(memory_space=pl.ANY),
        pl.BlockSpec(memory_space=pl.ANY),
        pl.BlockSpec((_KSMALL * _R, 1), lambda i: (0, 0)),
        pl.BlockSpec((_MBLK, _D), lambda i: (i, 0)),
    ],
    out_specs=pl.BlockSpec((_MBLK, _D), lambda i: (i, 0)),
    out_shape=jax.ShapeDtypeStruct((_D, _D), jnp.float32),
    scratch_shapes=[
        pltpu.VMEM((_KSMALL * _R, _D), jnp.float32),
        pltpu.VMEM((_KSMALL * _R, _D), jnp.float32),
        pltpu.SemaphoreType.DMA,
    ],
)


def kernel(q, corpus, A_all, B_all, W_base):
    wrow, ria8, cnt, idxs = _routing(q, corpus)
    a2d = A_all.reshape(_N * _R, _D)
    b2d = jnp.swapaxes(B_all, 1, 2).reshape(_N * _R, _D)

    def small_path():
        return _small_merge(idxs, cnt, a2d, b2d, wrow, W_base)

    def full_path():
        ag, bg = _make_sc_gather()(a2d, b2d, ria8.reshape(-1), cnt.reshape(-1))
        return _make_merge(_KPAD * _R)(bg, ag, wrow, cnt, W_base)

    return lax.cond(cnt[0, 0] <= _KSMALL, small_path, full_path)
